# baseline (speedup 1.0000x reference)
import functools

import jax
import jax.numpy as jnp
from jax import lax
from jax.experimental import pallas as pl
from jax.experimental.pallas import tpu as pltpu
from jax.experimental.pallas import tpu_sc as plsc

DIM = 32
BATCH = 16384
NB = 2 * BATCH
V = 1000000
NC = 2
NS = 16
PER_TILE = NB // NS
CHUNK = 128
N_CHUNK = PER_TILE // CHUNK
OUT_R = NB // CHUNK


def _sc_gather_body(nci_t, ma_t, idx_all, out_n, out_m,
                    buf_a, buf_b, idx_v, vals_v, sem_plane, sem_g):
    cid = lax.axis_index("c")
    sid = lax.axis_index("s")

    pltpu.sync_copy(idx_all.at[cid, sid], idx_v)

    bufs = (buf_a, buf_b)

    def plane_dma(f, buf):
        @pl.when(jnp.logical_and(sid == 0, cid == 0))
        def _(f=f):
            pltpu.async_copy(nci_t.at[f], buf, sem_plane)
        @pl.when(jnp.logical_and(sid == 0, cid == 1))
        def _(f=f):
            pltpu.async_copy(ma_t.at[f], buf, sem_plane)

    def plane_dma_drain(buf):
        @pl.when(sid == 0)
        def _():
            pltpu.make_async_copy(nci_t.at[0], buf, sem_plane).wait()

    plane_dma(0, bufs[0])

    for f in range(DIM):
        if f + 1 < DIM:
            plane_dma(f + 1, bufs[(f + 1) % 2])
        buf = bufs[f % 2]
        plane_dma_drain(buf)
        plsc.subcore_barrier()

        gathers = [
            pltpu.async_copy(buf.at[idx_v.at[j]], vals_v.at[j], sem_g)
            for j in range(N_CHUNK)
        ]
        for g in gathers:
            g.wait()

        dst = pl.ds(sid * N_CHUNK, N_CHUNK)
        @pl.when(cid == 0)
        def _(f=f, dst=dst):
            pltpu.sync_copy(vals_v, out_n.at[f, dst, :])
        @pl.when(cid == 1)
        def _(f=f, dst=dst):
            pltpu.sync_copy(vals_v, out_m.at[f, dst, :])
        plsc.subcore_barrier()


@jax.jit
def _sc_gather(nci_t, ma_t, idx_all):
    mesh = plsc.VectorSubcoreMesh(core_axis_name="c", subcore_axis_name="s")
    fn = pl.kernel(
        _sc_gather_body,
        out_type=[jax.ShapeDtypeStruct((DIM, OUT_R, CHUNK), jnp.float32),
                  jax.ShapeDtypeStruct((DIM, OUT_R, CHUNK), jnp.float32)],
        mesh=mesh,
        compiler_params=pltpu.CompilerParams(
            needs_layout_passes=False, use_tc_tiling_on_sc=True),
        scratch_types=[
            pltpu.VMEM_SHARED((V,), jnp.float32),
            pltpu.VMEM_SHARED((V,), jnp.float32),
            pltpu.VMEM((N_CHUNK, CHUNK), jnp.int32),
            pltpu.VMEM((N_CHUNK, CHUNK), jnp.float32),
            pltpu.SemaphoreType.DMA,
            pltpu.SemaphoreType.DMA,
        ],
    )
    return fn(nci_t, ma_t, idx_all)


def _tc_loss_body(n_ref, m_ref, out_ref):
    acc = jnp.zeros((OUT_R, CHUNK), jnp.float32)
    for f in range(DIM):
        d = n_ref[f] - m_ref[f]
        acc = acc + d * d
    p = acc[: OUT_R // 2]
    n = acc[OUT_R // 2:]
    p_loss = 1.0 / (1.0 + jnp.exp(p))
    n_loss = 1.0 / (1.0 + jnp.exp(n))
    pos_loss = jnp.sum(-jnp.log(p_loss))
    neg_loss = jnp.sum(-jnp.log(1.0 - n_loss))
    out_ref[0, 0] = pos_loss + neg_loss


@jax.jit
def _tc_loss(n_e, m_e):
    out = pl.pallas_call(
        _tc_loss_body,
        out_shape=jax.ShapeDtypeStruct((1, 1), jnp.float32),
        in_specs=[pl.BlockSpec(memory_space=pltpu.VMEM),
                  pl.BlockSpec(memory_space=pltpu.VMEM)],
        out_specs=pl.BlockSpec(memory_space=pltpu.SMEM),
    )(n_e, m_e)
    return out[0, 0]


def kernel(nci_ent_embeddings, ma_ent_embeddings, pos_n, pos_m, neg_n, neg_m):
    nci_t = nci_ent_embeddings.T
    ma_t = ma_ent_embeddings.T
    idx_n = jnp.concatenate([pos_n.astype(jnp.int32), neg_n.astype(jnp.int32)])
    idx_m = jnp.concatenate([pos_m.astype(jnp.int32), neg_m.astype(jnp.int32)])
    idx_all = jnp.stack([idx_n, idx_m]).reshape(2, NS, N_CHUNK, CHUNK)
    n_e, m_e = _sc_gather(nci_t, ma_t, idx_all)
    return _tc_loss(n_e, m_e)
